# Initial kernel scaffold; baseline (speedup 1.0000x reference)
import jax
import jax.numpy as jnp
from jax.experimental import pallas as pl


def kernel(x, base_W, A, B, router_W, lora_biases):
    raise NotImplementedError("write your pallas kernel here")


# speedup vs baseline: 9.5060x; 9.5060x over previous
import jax
import jax.numpy as jnp
from jax.experimental import pallas as pl
from jax.experimental.pallas import tpu as pltpu

_SEQ = 2048
_D = 1024
_R = 16
_ACT = 4
_SCALING = 8 / 4
_TILE_M = 256


def _body(x_ref, w_ref, a_ref, b_ref, rw_ref, bias_ref, o_ref):
    x = x_ref[...]

    z = jax.lax.dot_general(x, rw_ref[...], (((1,), (1,)), ((), ())),
                            preferred_element_type=jnp.float32)
    l = jax.nn.sigmoid(z) + bias_ref[...]
    m1 = jnp.max(l, axis=-1, keepdims=True)
    m2 = jnp.max(jnp.where(l < m1, l, -jnp.inf), axis=-1, keepdims=True)
    w = jnp.where(l >= m2, l, 0.0)
    gate = w * (_ACT / jnp.sum(w, axis=-1, keepdims=True))

    mid = jax.lax.dot_general(x, a_ref[...], (((1,), (1,)), ((), ())),
                              preferred_element_type=jnp.float32)
    lora = jax.lax.dot_general(mid * gate, b_ref[...], (((1,), (1,)), ((), ())),
                               preferred_element_type=jnp.float32)
    base = jax.lax.dot_general(x, w_ref[...], (((1,), (1,)), ((), ())),
                               preferred_element_type=jnp.float32)
    o_ref[...] = base + lora * _SCALING


def kernel(x, base_W, A, B, router_W, lora_biases):
    Bsz, S, Dm = x.shape
    n = Bsz * S
    xf = x.reshape(n, Dm)
    rw16 = jnp.repeat(router_W, _R // router_W.shape[0], axis=0)
    bias = lora_biases.reshape(1, _R)
    grid = (n // _TILE_M,)
    out = pl.pallas_call(
        _body,
        grid=grid,
        in_specs=[
            pl.BlockSpec((_TILE_M, Dm), lambda i: (i, 0)),
            pl.BlockSpec((Dm, Dm), lambda i: (0, 0)),
            pl.BlockSpec((_R, Dm), lambda i: (0, 0)),
            pl.BlockSpec((Dm, _R), lambda i: (0, 0)),
            pl.BlockSpec((_R, Dm), lambda i: (0, 0)),
            pl.BlockSpec((1, _R), lambda i: (0, 0)),
        ],
        out_specs=pl.BlockSpec((_TILE_M, Dm), lambda i: (i, 0)),
        out_shape=jax.ShapeDtypeStruct((n, Dm), jnp.float32),
    )(xf, base_W, A, B, rw16, bias)
    return out.reshape(Bsz, S, Dm)
